# initial kernel scaffold (unmeasured)
import functools

import jax
import jax.numpy as jnp
from jax import lax
from jax.experimental import pallas as pl
from jax.experimental.pallas import tpu as pltpu

N_DEV = 8
M_PER = 512
N_COL = 2048

PAYLOAD = jnp.float32


def kernel(x, w_mat, scale_x, scale_w):
    def body(x_ref, w_ref, sx_ref, sw_ref, out_ref,
             send_buf, recv_buf, send_sems, recv_sems):
        me = lax.axis_index("i")
        right = lax.rem(me + 1, N_DEV)
        left = lax.rem(me + N_DEV - 1, N_DEV)

        def partial(c):
            xs = x_ref[pl.ds(c * M_PER, M_PER), :]
            return jnp.dot(xs, w_ref[:, :], preferred_element_type=jnp.float32)

        barrier_sem = pltpu.get_barrier_semaphore()
        for nbr in (left, right):
            pl.semaphore_signal(
                barrier_sem, inc=1,
                device_id=(nbr,), device_id_type=pl.DeviceIdType.MESH,
            )
        pl.semaphore_wait(barrier_sem, 2)

        c0 = lax.rem(me + N_DEV - 1, N_DEV)
        send_buf[0] = partial(c0).astype(PAYLOAD)

        for h in range(N_DEV - 1):
            send_slot = h % 2
            rdma = pltpu.make_async_remote_copy(
                src_ref=send_buf.at[send_slot],
                dst_ref=recv_buf.at[h],
                send_sem=send_sems.at[send_slot],
                recv_sem=recv_sems.at[h],
                device_id=(right,),
                device_id_type=pl.DeviceIdType.MESH,
            )
            rdma.start()
            c = lax.rem(me + 2 * N_DEV - 2 - h, N_DEV)
            p = partial(c)
            rdma.wait()
            acc = recv_buf[h].astype(jnp.float32) + p
            if h < N_DEV - 2:
                send_buf[(h + 1) % 2] = acc.astype(PAYLOAD)
            else:
                s = sx_ref[0] * sw_ref[0]
                out_ref[:, :] = acc * s

    return pl.pallas_call(
        body,
        out_shape=jax.ShapeDtypeStruct((M_PER, N_COL), jnp.float32),
        in_specs=[
            pl.BlockSpec(memory_space=pltpu.VMEM),
            pl.BlockSpec(memory_space=pltpu.VMEM),
            pl.BlockSpec(memory_space=pltpu.SMEM),
            pl.BlockSpec(memory_space=pltpu.SMEM),
        ],
        out_specs=pl.BlockSpec(memory_space=pltpu.VMEM),
        scratch_shapes=[
            pltpu.VMEM((2, M_PER, N_COL), PAYLOAD),
            pltpu.VMEM((N_DEV - 1, M_PER, N_COL), PAYLOAD),
            pltpu.SemaphoreType.DMA((2,)),
            pltpu.SemaphoreType.DMA((N_DEV - 1,)),
        ],
        compiler_params=pltpu.CompilerParams(collective_id=0),
    )(x, w_mat, scale_x, scale_w)


# baseline (device time: 191066 ns/iter reference)
import functools

import jax
import jax.numpy as jnp
from jax import lax
from jax.experimental import pallas as pl
from jax.experimental.pallas import tpu as pltpu

N_DEV = 8
M_PER = 512
N_COL = 2048

PAYLOAD = jnp.bfloat16


def kernel(x, w_mat, scale_x, scale_w):
    def body(x_ref, w_ref, sx_ref, sw_ref, out_ref,
             send_buf, recv_buf, send_sems, recv_sems):
        me = lax.axis_index("i")
        right = lax.rem(me + 1, N_DEV)
        left = lax.rem(me + N_DEV - 1, N_DEV)

        def partial(c):
            xs = x_ref[pl.ds(c * M_PER, M_PER), :]
            return jnp.dot(xs, w_ref[:, :], preferred_element_type=jnp.float32)

        barrier_sem = pltpu.get_barrier_semaphore()
        for nbr in (left, right):
            pl.semaphore_signal(
                barrier_sem, inc=1,
                device_id=(nbr,), device_id_type=pl.DeviceIdType.MESH,
            )
        pl.semaphore_wait(barrier_sem, 2)

        c0 = lax.rem(me + N_DEV - 1, N_DEV)
        send_buf[0] = partial(c0).astype(PAYLOAD)

        for h in range(N_DEV - 1):
            send_slot = h % 2
            rdma = pltpu.make_async_remote_copy(
                src_ref=send_buf.at[send_slot],
                dst_ref=recv_buf.at[h],
                send_sem=send_sems.at[send_slot],
                recv_sem=recv_sems.at[h],
                device_id=(right,),
                device_id_type=pl.DeviceIdType.MESH,
            )
            rdma.start()
            c = lax.rem(me + 2 * N_DEV - 2 - h, N_DEV)
            p = partial(c)
            rdma.wait()
            acc = recv_buf[h].astype(jnp.float32) + p
            if h < N_DEV - 2:
                send_buf[(h + 1) % 2] = acc.astype(PAYLOAD)
            else:
                s = sx_ref[0] * sw_ref[0]
                out_ref[:, :] = acc * s

    return pl.pallas_call(
        body,
        out_shape=jax.ShapeDtypeStruct((M_PER, N_COL), jnp.float32),
        in_specs=[
            pl.BlockSpec(memory_space=pltpu.VMEM),
            pl.BlockSpec(memory_space=pltpu.VMEM),
            pl.BlockSpec(memory_space=pltpu.SMEM),
            pl.BlockSpec(memory_space=pltpu.SMEM),
        ],
        out_specs=pl.BlockSpec(memory_space=pltpu.VMEM),
        scratch_shapes=[
            pltpu.VMEM((2, M_PER, N_COL), PAYLOAD),
            pltpu.VMEM((N_DEV - 1, M_PER, N_COL), PAYLOAD),
            pltpu.SemaphoreType.DMA((2,)),
            pltpu.SemaphoreType.DMA((N_DEV - 1,)),
        ],
        compiler_params=pltpu.CompilerParams(
            collective_id=0,
            vmem_limit_bytes=100 * 1024 * 1024,
        ),
    )(x, w_mat, scale_x, scale_w)


# device time: 115150 ns/iter; 1.6593x vs baseline; 1.6593x over previous
import jax
import jax.numpy as jnp
from jax import lax
from jax.experimental import pallas as pl
from jax.experimental.pallas import tpu as pltpu

N_DEV = 8
M_PER = 512
N_COL = 2048
N_HALF = N_COL // 2

PAYLOAD = jnp.bfloat16


def kernel(x, w_mat, scale_x, scale_w):
    def body(x_ref, w_ref, sx_ref, sw_ref, out_ref,
             send_a, send_b, recv_a, recv_b,
             send_sems_a, send_sems_b, recv_sems_a, recv_sems_b):
        me = lax.axis_index("i")
        right = lax.rem(me + 1, N_DEV)
        left = lax.rem(me + N_DEV - 1, N_DEV)

        def partial_a(c):
            xs = x_ref[pl.ds(c * M_PER, M_PER), :].astype(jnp.bfloat16)
            ws = w_ref[:, :N_HALF].astype(jnp.bfloat16)
            return jnp.dot(xs, ws, preferred_element_type=jnp.float32)

        def partial_b(c):
            xs = x_ref[pl.ds(c * M_PER, M_PER), :].astype(jnp.bfloat16)
            ws = w_ref[:, N_HALF:].astype(jnp.bfloat16)
            return jnp.dot(xs, ws, preferred_element_type=jnp.float32)

        barrier_sem = pltpu.get_barrier_semaphore()
        for nbr in (left, right):
            pl.semaphore_signal(
                barrier_sem, inc=1,
                device_id=(nbr,), device_id_type=pl.DeviceIdType.MESH,
            )
        pl.semaphore_wait(barrier_sem, 2)

        send_a[0] = partial_a(lax.rem(me + N_DEV - 1, N_DEV)).astype(PAYLOAD)
        send_b[0] = partial_b(lax.rem(me + 1, N_DEV)).astype(PAYLOAD)

        for h in range(N_DEV - 1):
            slot = h % 2
            rdma_a = pltpu.make_async_remote_copy(
                src_ref=send_a.at[slot],
                dst_ref=recv_a.at[h],
                send_sem=send_sems_a.at[slot],
                recv_sem=recv_sems_a.at[h],
                device_id=(right,),
                device_id_type=pl.DeviceIdType.MESH,
            )
            rdma_b = pltpu.make_async_remote_copy(
                src_ref=send_b.at[slot],
                dst_ref=recv_b.at[h],
                send_sem=send_sems_b.at[slot],
                recv_sem=recv_sems_b.at[h],
                device_id=(left,),
                device_id_type=pl.DeviceIdType.MESH,
            )
            rdma_a.start()
            rdma_b.start()
            ca = lax.rem(me + 2 * N_DEV - 2 - h, N_DEV)
            cb = lax.rem(me + 2 + h, N_DEV)
            pa = partial_a(ca)
            pb = partial_b(cb)
            rdma_a.wait()
            rdma_b.wait()
            acc_a = recv_a[h].astype(jnp.float32) + pa
            acc_b = recv_b[h].astype(jnp.float32) + pb
            if h < N_DEV - 2:
                send_a[(h + 1) % 2] = acc_a.astype(PAYLOAD)
                send_b[(h + 1) % 2] = acc_b.astype(PAYLOAD)
            else:
                s = sx_ref[0] * sw_ref[0]
                out_ref[:, :N_HALF] = acc_a * s
                out_ref[:, N_HALF:] = acc_b * s

    return pl.pallas_call(
        body,
        out_shape=jax.ShapeDtypeStruct((M_PER, N_COL), jnp.float32),
        in_specs=[
            pl.BlockSpec(memory_space=pltpu.VMEM),
            pl.BlockSpec(memory_space=pltpu.VMEM),
            pl.BlockSpec(memory_space=pltpu.SMEM),
            pl.BlockSpec(memory_space=pltpu.SMEM),
        ],
        out_specs=pl.BlockSpec(memory_space=pltpu.VMEM),
        scratch_shapes=[
            pltpu.VMEM((2, M_PER, N_HALF), PAYLOAD),
            pltpu.VMEM((2, M_PER, N_HALF), PAYLOAD),
            pltpu.VMEM((N_DEV - 1, M_PER, N_HALF), PAYLOAD),
            pltpu.VMEM((N_DEV - 1, M_PER, N_HALF), PAYLOAD),
            pltpu.SemaphoreType.DMA((2,)),
            pltpu.SemaphoreType.DMA((2,)),
            pltpu.SemaphoreType.DMA((N_DEV - 1,)),
            pltpu.SemaphoreType.DMA((N_DEV - 1,)),
        ],
        compiler_params=pltpu.CompilerParams(
            collective_id=0,
            vmem_limit_bytes=100 * 1024 * 1024,
        ),
    )(x, w_mat, scale_x, scale_w)


# device time: 97544 ns/iter; 1.9588x vs baseline; 1.1805x over previous
import jax
import jax.numpy as jnp
from jax import lax
from jax.experimental import pallas as pl
from jax.experimental.pallas import tpu as pltpu

N_DEV = 8
M_PER = 512
N_COL = 2048
N_HALF = N_COL // 2
N_SUB = 4
W_SUB = N_HALF // N_SUB

PAYLOAD = jnp.bfloat16


def kernel(x, w_mat, scale_x, scale_w):
    def body(x_ref, w_ref, sx_ref, sw_ref, out_ref,
             send_a, send_b, recv_a, recv_b,
             send_sems_a, send_sems_b, recv_sems_a, recv_sems_b):
        me = lax.axis_index("i")
        right = lax.rem(me + 1, N_DEV)
        left = lax.rem(me + N_DEV - 1, N_DEV)

        def partial(c, lo, width):
            xs = x_ref[pl.ds(c * M_PER, M_PER), :].astype(jnp.bfloat16)
            ws = w_ref[:, lo:lo + width].astype(jnp.bfloat16)
            return jnp.dot(xs, ws, preferred_element_type=jnp.float32)

        def rdma(buffers, h, s, target):
            send_buf, recv_buf, send_sems, recv_sems = buffers
            return pltpu.make_async_remote_copy(
                src_ref=send_buf.at[h % 2, s],
                dst_ref=recv_buf.at[h, s],
                send_sem=send_sems.at[h % 2, s],
                recv_sem=recv_sems.at[h, s],
                device_id=(target,),
                device_id_type=pl.DeviceIdType.MESH,
            )

        bufs_a = (send_a, recv_a, send_sems_a, recv_sems_a)
        bufs_b = (send_b, recv_b, send_sems_b, recv_sems_b)

        barrier_sem = pltpu.get_barrier_semaphore()
        for nbr in (left, right):
            pl.semaphore_signal(
                barrier_sem, inc=1,
                device_id=(nbr,), device_id_type=pl.DeviceIdType.MESH,
            )
        pl.semaphore_wait(barrier_sem, 2)

        seed_a = partial(lax.rem(me + N_DEV - 1, N_DEV), 0, N_HALF)
        seed_b = partial(lax.rem(me + 1, N_DEV), N_HALF, N_HALF)
        for s in range(N_SUB):
            lo, hi = s * W_SUB, (s + 1) * W_SUB
            send_a[0, s] = seed_a[:, lo:hi].astype(PAYLOAD)
            rdma(bufs_a, 0, s, right).start()
            send_b[0, s] = seed_b[:, lo:hi].astype(PAYLOAD)
            rdma(bufs_b, 0, s, left).start()

        s_out = sx_ref[0] * sw_ref[0]
        for h in range(N_DEV - 1):
            ca = lax.rem(me + 2 * N_DEV - 2 - h, N_DEV)
            cb = lax.rem(me + 2 + h, N_DEV)
            pa = partial(ca, 0, N_HALF)
            pb = partial(cb, N_HALF, N_HALF)
            for s in range(N_SUB):
                lo, hi = s * W_SUB, (s + 1) * W_SUB
                rdma(bufs_a, h, s, right).wait()
                acc_a = recv_a[h, s].astype(jnp.float32) + pa[:, lo:hi]
                rdma(bufs_b, h, s, left).wait()
                acc_b = recv_b[h, s].astype(jnp.float32) + pb[:, lo:hi]
                if h < N_DEV - 2:
                    send_a[(h + 1) % 2, s] = acc_a.astype(PAYLOAD)
                    rdma(bufs_a, h + 1, s, right).start()
                    send_b[(h + 1) % 2, s] = acc_b.astype(PAYLOAD)
                    rdma(bufs_b, h + 1, s, left).start()
                else:
                    out_ref[:, lo:hi] = acc_a * s_out
                    out_ref[:, N_HALF + lo:N_HALF + hi] = acc_b * s_out

    return pl.pallas_call(
        body,
        out_shape=jax.ShapeDtypeStruct((M_PER, N_COL), jnp.float32),
        in_specs=[
            pl.BlockSpec(memory_space=pltpu.VMEM),
            pl.BlockSpec(memory_space=pltpu.VMEM),
            pl.BlockSpec(memory_space=pltpu.SMEM),
            pl.BlockSpec(memory_space=pltpu.SMEM),
        ],
        out_specs=pl.BlockSpec(memory_space=pltpu.VMEM),
        scratch_shapes=[
            pltpu.VMEM((2, N_SUB, M_PER, W_SUB), PAYLOAD),
            pltpu.VMEM((2, N_SUB, M_PER, W_SUB), PAYLOAD),
            pltpu.VMEM((N_DEV - 1, N_SUB, M_PER, W_SUB), PAYLOAD),
            pltpu.VMEM((N_DEV - 1, N_SUB, M_PER, W_SUB), PAYLOAD),
            pltpu.SemaphoreType.DMA((2, N_SUB)),
            pltpu.SemaphoreType.DMA((2, N_SUB)),
            pltpu.SemaphoreType.DMA((N_DEV - 1, N_SUB)),
            pltpu.SemaphoreType.DMA((N_DEV - 1, N_SUB)),
        ],
        compiler_params=pltpu.CompilerParams(
            collective_id=0,
            vmem_limit_bytes=100 * 1024 * 1024,
        ),
    )(x, w_mat, scale_x, scale_w)


# device time: 76208 ns/iter; 2.5072x vs baseline; 1.2800x over previous
import jax
import jax.numpy as jnp
from jax import lax
from jax.experimental import pallas as pl
from jax.experimental.pallas import tpu as pltpu

N_DEV = 8
M_PER = 512
N_COL = 2048
N_HALF = N_COL // 2
N_SUB = 4
W_SUB = N_HALF // N_SUB

PAYLOAD = jnp.bfloat16

STREAMS = (((4, 3, 1), 0), ((1, 3, 4), N_HALF))


def kernel(x, w_mat, scale_x, scale_w):
    def body(x_ref, w_ref, sx_ref, sw_ref, out_ref,
             sbuf_a, rbuf_a, ssem_a, rsem_a,
             sbuf_b, rbuf_b, ssem_b, rsem_b):
        me = lax.axis_index("i")
        bufs = (
            (sbuf_a, rbuf_a, ssem_a, rsem_a),
            (sbuf_b, rbuf_b, ssem_b, rsem_b),
        )

        def partner(mask):
            return jnp.bitwise_xor(me, mask)

        def partial(c, lo):
            xs = x_ref[pl.ds(c * M_PER, M_PER), :].astype(jnp.bfloat16)
            ws = w_ref[:, lo:lo + N_HALF].astype(jnp.bfloat16)
            return jnp.dot(xs, ws, preferred_element_type=jnp.float32)

        def rdma(t, slot, s, mask):
            sbuf, rbuf, ssem, rsem = bufs[t]
            return pltpu.make_async_remote_copy(
                src_ref=sbuf.at[slot, s],
                dst_ref=rbuf.at[slot, s],
                send_sem=ssem.at[slot, s],
                recv_sem=rsem.at[slot, s],
                device_id=(partner(mask),),
                device_id_type=pl.DeviceIdType.MESH,
            )

        barrier_sem = pltpu.get_barrier_semaphore()
        for mask in (1, 3, 4):
            pl.semaphore_signal(
                barrier_sem, inc=1,
                device_id=(partner(mask),), device_id_type=pl.DeviceIdType.MESH,
            )
        pl.semaphore_wait(barrier_sem, 3)

        for k in range(4):
            for t, ((m1, m2, m3), lo) in enumerate(STREAMS):
                e = (m1 ^ m2 ^ m3, m1 ^ m2, m1 ^ m3, m1)[k]
                p = partial(jnp.bitwise_xor(me, e), lo)
                sbuf = bufs[t][0]
                for s in range(N_SUB):
                    sbuf[k, s] = p[:, s * W_SUB:(s + 1) * W_SUB].astype(PAYLOAD)
                    rdma(t, k, s, m1).start()

        for j in range(2):
            for t, ((m1, m2, m3), lo) in enumerate(STREAMS):
                e = (m2 ^ m3, m2)[j]
                p = partial(jnp.bitwise_xor(me, e), lo)
                sbuf, rbuf = bufs[t][0], bufs[t][1]
                for s in range(N_SUB):
                    rdma(t, j, s, m1).wait()
                    acc = rbuf[j, s].astype(jnp.float32) \
                        + p[:, s * W_SUB:(s + 1) * W_SUB]
                    sbuf[4 + j, s] = acc.astype(PAYLOAD)
                    rdma(t, 4 + j, s, m2).start()

        for t, ((m1, m2, m3), lo) in enumerate(STREAMS):
            p = partial(jnp.bitwise_xor(me, m3), lo)
            sbuf, rbuf = bufs[t][0], bufs[t][1]
            for s in range(N_SUB):
                rdma(t, 2, s, m1).wait()
                rdma(t, 4, s, m2).wait()
                acc = rbuf[2, s].astype(jnp.float32) \
                    + rbuf[4, s].astype(jnp.float32) \
                    + p[:, s * W_SUB:(s + 1) * W_SUB]
                sbuf[6, s] = acc.astype(PAYLOAD)
                rdma(t, 6, s, m3).start()

        s_out = sx_ref[0] * sw_ref[0]
        for t, ((m1, m2, m3), lo) in enumerate(STREAMS):
            p = partial(me, lo)
            rbuf = bufs[t][1]
            for s in range(N_SUB):
                rdma(t, 3, s, m1).wait()
                rdma(t, 5, s, m2).wait()
                rdma(t, 6, s, m3).wait()
                acc = rbuf[3, s].astype(jnp.float32) \
                    + rbuf[5, s].astype(jnp.float32) \
                    + rbuf[6, s].astype(jnp.float32) \
                    + p[:, s * W_SUB:(s + 1) * W_SUB]
                out_ref[:, lo + s * W_SUB:lo + (s + 1) * W_SUB] = acc * s_out

    comm = pltpu.VMEM((7, N_SUB, M_PER, W_SUB), PAYLOAD)
    sems = pltpu.SemaphoreType.DMA((7, N_SUB))
    return pl.pallas_call(
        body,
        out_shape=jax.ShapeDtypeStruct((M_PER, N_COL), jnp.float32),
        in_specs=[
            pl.BlockSpec(memory_space=pltpu.VMEM),
            pl.BlockSpec(memory_space=pltpu.VMEM),
            pl.BlockSpec(memory_space=pltpu.SMEM),
            pl.BlockSpec(memory_space=pltpu.SMEM),
        ],
        out_specs=pl.BlockSpec(memory_space=pltpu.VMEM),
        scratch_shapes=[comm, comm, sems, sems, comm, comm, sems, sems],
        compiler_params=pltpu.CompilerParams(
            collective_id=0,
            vmem_limit_bytes=100 * 1024 * 1024,
        ),
    )(x, w_mat, scale_x, scale_w)
